# Initial kernel scaffold; baseline (speedup 1.0000x reference)
import jax
import jax.numpy as jnp
from jax.experimental import pallas as pl


def kernel(pos_u, pos_v, neg_v, u_table, v_table):
    raise NotImplementedError("write your pallas kernel here")


# speedup vs baseline: 4.9444x; 4.9444x over previous
import functools

import jax
import jax.numpy as jnp
from jax import lax
from jax.experimental import pallas as pl
from jax.experimental.pallas import tpu as pltpu
from jax.experimental.pallas import tpu_sc as plsc

B = 16384
D = 64
K = 20
NC = 2
NS = 16
NW = NC * NS
EPW = B // NW
CH = 32
NCH = EPW // CH
DOTS = CH * (K + 1)
SCW = NCH * DOTS
TOT = NW * SCW
GQ = 6
GN = DOTS // GQ


def _sc_body(pos_u_hbm, v_idx_hbm, u_table_hbm, v_table_hbm, out_hbm,
             u_idx, v_idx, emb_u, rows, cumbuf, scores, sem):
    wid = lax.axis_index("s") * NC + lax.axis_index("c")
    pltpu.sync_copy(pos_u_hbm.at[pl.ds(wid * EPW, EPW)], u_idx)
    rowstart = lax.iota(jnp.int32, 16) * 16

    def chunk_body(c, carry):
        gchunk = wid * NCH + c
        pltpu.sync_copy(v_idx_hbm.at[gchunk], v_idx)
        cp_u = pltpu.async_copy(
            u_table_hbm.at[u_idx.at[pl.ds(c * CH, CH)]], emb_u, sem)
        cps = [
            pltpu.async_copy(
                v_table_hbm.at[v_idx.at[pl.ds(q * GN, GN)]],
                rows.at[pl.ds(q * GN, GN)], sem)
            for q in range(GQ)
        ]
        cp_u.wait()
        for cp in cps:
            cp.wait()

        def pos_body(e, carry2):
            acc = emb_u[e, pl.ds(0, 16)] * rows[e, pl.ds(0, 16)]
            for q in range(1, 4):
                acc = acc + emb_u[e, pl.ds(q * 16, 16)] * rows[e, pl.ds(q * 16, 16)]
            cumbuf[pl.ds(e * 16, 16)] = acc
            return carry2

        lax.fori_loop(0, CH, pos_body, 0, unroll=4)

        def neg_body(e, carry2):
            u0 = emb_u[e, pl.ds(0, 16)]
            u1 = emb_u[e, pl.ds(16, 16)]
            u2 = emb_u[e, pl.ds(32, 16)]
            u3 = emb_u[e, pl.ds(48, 16)]
            rbase = CH + e * K
            for k in range(K):
                r = rbase + k
                acc = (u0 * rows[r, pl.ds(0, 16)]
                       + u1 * rows[r, pl.ds(16, 16)]
                       + u2 * rows[r, pl.ds(32, 16)]
                       + u3 * rows[r, pl.ds(48, 16)])
                cumbuf[pl.ds(r * 16, 16)] = acc
            return carry2

        lax.fori_loop(0, CH, neg_body, 0)

        sbase = c * DOTS

        def fin_body(g, carry2):
            base = g * 256
            t = plsc.load_gather(cumbuf, [base + rowstart])
            for j in range(1, 16):
                t = t + plsc.load_gather(cumbuf, [base + rowstart + j])
            scores[pl.ds(sbase + g * 16, 16)] = t
            return carry2

        lax.fori_loop(0, DOTS // 16, fin_body, 0, unroll=4)
        return carry

    lax.fori_loop(0, NCH, chunk_body, 0)
    pltpu.sync_copy(scores, out_hbm.at[pl.ds(wid * SCW, SCW)])


_sc_scores = pl.kernel(
    _sc_body,
    out_type=jax.ShapeDtypeStruct((TOT,), jnp.float32),
    mesh=plsc.VectorSubcoreMesh(
        core_axis_name="c", subcore_axis_name="s",
        num_cores=NC, num_subcores=NS),
    compiler_params=pltpu.CompilerParams(
        needs_layout_passes=False, use_tc_tiling_on_sc=False),
    scratch_types=[
        pltpu.VMEM((EPW,), jnp.int32),
        pltpu.VMEM((DOTS,), jnp.int32),
        pltpu.VMEM((CH, D), jnp.float32),
        pltpu.VMEM((DOTS, D), jnp.float32),
        pltpu.VMEM((DOTS * 16,), jnp.float32),
        pltpu.VMEM((SCW,), jnp.float32),
        pltpu.SemaphoreType.DMA,
    ],
)

_TC_ROWS = TOT // 128


def _tc_body(x_ref, o_ref):
    x = x_ref[...]
    n = (lax.broadcasted_iota(jnp.int32, (_TC_ROWS, 128), 0) * 128
         + lax.broadcasted_iota(jnp.int32, (_TC_ROWS, 128), 1))
    r = n % DOTS
    is_pos = r < CH
    xc = jnp.clip(x, -10.0, 10.0)
    t = jnp.where(is_pos, -xc, xc)
    term = jnp.log1p(jnp.exp(t))
    pos_mean = jnp.sum(jnp.where(is_pos, term, 0.0)) * (1.0 / B)
    neg_mean = jnp.sum(jnp.where(is_pos, 0.0, term)) * (1.0 / (B * K))
    lane = lax.broadcasted_iota(jnp.int32, (1, 128), 1)
    o_ref[...] = jnp.where(lane == 0, pos_mean,
                           jnp.where(lane == 1, neg_mean, 0.0))


_tc_loss = pl.pallas_call(
    _tc_body,
    out_shape=jax.ShapeDtypeStruct((1, 128), jnp.float32),
)


def kernel(pos_u, pos_v, neg_v, u_table, v_table):
    v_idx = jnp.concatenate(
        [pos_v.reshape(B // CH, CH), neg_v.reshape(B // CH, CH * K)], axis=1)
    scores = _sc_scores(pos_u, v_idx, u_table, v_table)
    sums = _tc_loss(scores.reshape(_TC_ROWS, 128))
    a = sums[0, 0]
    b = sums[0, 1]
    return (a + b, a, b)
